# Initial kernel scaffold; baseline (speedup 1.0000x reference)
import jax
import jax.numpy as jnp
from jax.experimental import pallas as pl


def kernel(x, gcn_W0, gcn_b0, gcn_W1, gcn_b1, gcn_W2, gcn_b2, mlp_W, mlp_b, pw2_W, pw2_b, pw1_W, pw1_b, edge_index):
    raise NotImplementedError("write your pallas kernel here")


# speedup vs baseline: 9.7812x; 9.7812x over previous
import functools

import jax
import jax.numpy as jnp
from jax import lax
from jax.experimental import pallas as pl
from jax.experimental.pallas import tpu as pltpu
from jax.experimental.pallas import tpu_sc as plsc

N_NODES = 10000
N_EDGES = 320000
D_IN = 771
H = 128
ALPHA = 0.1
FRAC = 0.5

NUM_CORES = 2
NUM_SUBCORES = 16
NUM_TILES = NUM_CORES * NUM_SUBCORES
EDGES_PER_TILE = N_EDGES // NUM_TILES
CHUNK = 80
NUM_CHUNKS = EDGES_PER_TILE // CHUNK
ROWS_PER_SUBCORE = 624
ROWS_TAIL = N_NODES - NUM_SUBCORES * ROWS_PER_SUBCORE

_MESH = plsc.VectorSubcoreMesh(core_axis_name="c", subcore_axis_name="s")


@functools.partial(
    pl.kernel,
    mesh=_MESH,
    out_type=jax.ShapeDtypeStruct((NUM_CORES, N_NODES, H), jnp.float32),
    scratch_types=[
        pltpu.VMEM((CHUNK,), jnp.int32),
        pltpu.VMEM((CHUNK, H), jnp.float32),
        pltpu.VMEM_SHARED((N_NODES, H), jnp.float32),
    ],
)
def _sc_degree(dst_hbm, zeros_hbm, ones_hbm, out_hbm, didx, ones_v, acc):
    c = lax.axis_index("c")
    s = lax.axis_index("s")

    pltpu.sync_copy(ones_hbm, ones_v)

    row0 = s * ROWS_PER_SUBCORE
    tail0 = NUM_SUBCORES * ROWS_PER_SUBCORE
    pltpu.sync_copy(zeros_hbm.at[pl.ds(0, ROWS_PER_SUBCORE)],
                    acc.at[pl.ds(row0, ROWS_PER_SUBCORE)])

    @pl.when(s == NUM_SUBCORES - 1)
    def _():
        pltpu.sync_copy(zeros_hbm.at[pl.ds(0, ROWS_TAIL)],
                        acc.at[pl.ds(tail0, ROWS_TAIL)])

    plsc.subcore_barrier()

    tile_base = (c * NUM_SUBCORES + s) * EDGES_PER_TILE

    def body(i, _):
        base = tile_base + i * CHUNK
        pltpu.sync_copy(dst_hbm.at[pl.ds(base, CHUNK)], didx)
        pltpu.sync_copy(ones_v, acc.at[didx], add=True)
        return _
    lax.fori_loop(0, NUM_CHUNKS, body, None)

    plsc.subcore_barrier()
    pltpu.sync_copy(acc.at[pl.ds(row0, ROWS_PER_SUBCORE)],
                    out_hbm.at[c, pl.ds(row0, ROWS_PER_SUBCORE)])

    @pl.when(s == NUM_SUBCORES - 1)
    def _():
        pltpu.sync_copy(acc.at[pl.ds(tail0, ROWS_TAIL)],
                        out_hbm.at[c, pl.ds(tail0, ROWS_TAIL)])


@functools.partial(
    pl.kernel,
    mesh=_MESH,
    out_type=jax.ShapeDtypeStruct((NUM_CORES, N_NODES, H), jnp.float32),
    scratch_types=[
        pltpu.VMEM((CHUNK,), jnp.int32),
        pltpu.VMEM((CHUNK,), jnp.int32),
        pltpu.VMEM((CHUNK, H), jnp.float32),
        pltpu.VMEM_SHARED((N_NODES, H), jnp.float32),
        pltpu.SemaphoreType.DMA,
    ],
)
def _sc_edge_sum(g_hbm, src_hbm, dst_hbm, zeros_hbm, out_hbm,
                 sidx, didx, rows, acc, sem):
    c = lax.axis_index("c")
    s = lax.axis_index("s")

    row0 = s * ROWS_PER_SUBCORE
    tail0 = NUM_SUBCORES * ROWS_PER_SUBCORE
    pltpu.sync_copy(zeros_hbm.at[pl.ds(0, ROWS_PER_SUBCORE)],
                    acc.at[pl.ds(row0, ROWS_PER_SUBCORE)])

    @pl.when(s == NUM_SUBCORES - 1)
    def _():
        pltpu.sync_copy(zeros_hbm.at[pl.ds(0, ROWS_TAIL)],
                        acc.at[pl.ds(tail0, ROWS_TAIL)])

    plsc.subcore_barrier()

    tile_base = (c * NUM_SUBCORES + s) * EDGES_PER_TILE

    def body(i, _):
        base = tile_base + i * CHUNK
        pltpu.sync_copy(src_hbm.at[pl.ds(base, CHUNK)], sidx)
        pltpu.sync_copy(dst_hbm.at[pl.ds(base, CHUNK)], didx)
        pltpu.async_copy(g_hbm.at[sidx], rows, sem).wait()
        pltpu.sync_copy(rows, acc.at[didx], add=True)
        return _
    lax.fori_loop(0, NUM_CHUNKS, body, None)

    plsc.subcore_barrier()
    pltpu.sync_copy(acc.at[pl.ds(row0, ROWS_PER_SUBCORE)],
                    out_hbm.at[c, pl.ds(row0, ROWS_PER_SUBCORE)])

    @pl.when(s == NUM_SUBCORES - 1)
    def _():
        pltpu.sync_copy(acc.at[pl.ds(tail0, ROWS_TAIL)],
                        out_hbm.at[c, pl.ds(tail0, ROWS_TAIL)])


_R = 1000


def _tc_front_body(x_ref, w0_ref, mlpw_ref, mlpb_ref, pw1w_ref, pw1b_ref,
                   degp_ref, g0_ref, proj1_ref, dinv_ref):
    x = x_ref[...]
    deg = degp_ref[0, :, 0] + degp_ref[1, :, 0] + 1.0
    dinv = lax.rsqrt(jnp.maximum(deg, 1.0))[:, None]
    dinv_ref[...] = dinv
    g0_ref[...] = jnp.dot(x, w0_ref[...], preferred_element_type=jnp.float32) * dinv
    r1 = jnp.maximum(
        jnp.dot(x, mlpw_ref[...], preferred_element_type=jnp.float32)
        + mlpb_ref[...], 0.0)
    t = jnp.dot(r1, pw1w_ref[...], preferred_element_type=jnp.float32) + pw1b_ref[...]
    proj1_ref[...] = jnp.where(t > 0, t, ALPHA * t)


def _tc_mid_body(p_ref, g_ref, dinv_ref, b_ref, w_ref, gout_ref):
    dinv = dinv_ref[...]
    h = jnp.maximum((p_ref[0] + p_ref[1] + g_ref[...]) * dinv + b_ref[...], 0.0)
    gout_ref[...] = jnp.dot(h, w_ref[...], preferred_element_type=jnp.float32) * dinv


def _tc_final_body(p_ref, g_ref, dinv_ref, b_ref, pw2w_ref, pw2b_ref,
                   proj1_ref, out_ref):
    dinv = dinv_ref[...]
    rna2d = jnp.maximum((p_ref[0] + p_ref[1] + g_ref[...]) * dinv + b_ref[...], 0.0)
    t = jnp.dot(rna2d, pw2w_ref[...], preferred_element_type=jnp.float32) + pw2b_ref[...]
    proj2 = jnp.where(t > 0, t, ALPHA * t)
    out_ref[...] = FRAC * proj2 + (1.0 - FRAC) * proj1_ref[...]


def _row_spec(width):
    return pl.BlockSpec((_R, width), lambda i: (i, 0))


def _full_spec(shape):
    nd = len(shape)
    return pl.BlockSpec(shape, lambda i: (0,) * nd)


def _part_spec(width):
    return pl.BlockSpec((NUM_CORES, _R, width), lambda i: (0, i, 0))


_GRID = (N_NODES // _R,)


def _tc_front(x, w0, mlpw, mlpb, pw1w, pw1b, degp):
    return pl.pallas_call(
        _tc_front_body,
        grid=_GRID,
        in_specs=[
            _row_spec(D_IN),
            _full_spec((D_IN, H)),
            _full_spec((D_IN, H)),
            _full_spec((1, H)),
            _full_spec((H, H)),
            _full_spec((1, H)),
            _part_spec(H),
        ],
        out_specs=[_row_spec(H), _row_spec(H), _row_spec(1)],
        out_shape=[
            jax.ShapeDtypeStruct((N_NODES, H), jnp.float32),
            jax.ShapeDtypeStruct((N_NODES, H), jnp.float32),
            jax.ShapeDtypeStruct((N_NODES, 1), jnp.float32),
        ],
    )(x, w0, mlpw, mlpb, pw1w, pw1b, degp)


def _tc_mid(p, g, dinv, b, w):
    return pl.pallas_call(
        _tc_mid_body,
        grid=_GRID,
        in_specs=[
            _part_spec(H),
            _row_spec(H),
            _row_spec(1),
            _full_spec((1, H)),
            _full_spec((H, H)),
        ],
        out_specs=_row_spec(H),
        out_shape=jax.ShapeDtypeStruct((N_NODES, H), jnp.float32),
    )(p, g, dinv, b, w)


def _tc_final(p, g, dinv, b, pw2w, pw2b, proj1):
    return pl.pallas_call(
        _tc_final_body,
        grid=_GRID,
        in_specs=[
            _part_spec(H),
            _row_spec(H),
            _row_spec(1),
            _full_spec((1, H)),
            _full_spec((H, H)),
            _full_spec((1, H)),
            _row_spec(H),
        ],
        out_specs=_row_spec(H),
        out_shape=jax.ShapeDtypeStruct((N_NODES, H), jnp.float32),
    )(p, g, dinv, b, pw2w, pw2b, proj1)


def kernel(x, gcn_W0, gcn_b0, gcn_W1, gcn_b1, gcn_W2, gcn_b2,
           mlp_W, mlp_b, pw2_W, pw2_b, pw1_W, pw1_b, edge_index):
    src = edge_index[0].astype(jnp.int32)
    dst = edge_index[1].astype(jnp.int32)

    zerosH = jnp.zeros((ROWS_PER_SUBCORE, H), jnp.float32)
    onesH = jnp.ones((CHUNK, H), jnp.float32)

    degp = _sc_degree(dst, zerosH, onesH)

    g0, proj1, dinv = _tc_front(
        x, gcn_W0, mlp_W, mlp_b.reshape(1, H), pw1_W, pw1_b.reshape(1, H), degp)

    p0 = _sc_edge_sum(g0, src, dst, zerosH)
    g1 = _tc_mid(p0, g0, dinv, gcn_b0.reshape(1, H), gcn_W1)
    p1 = _sc_edge_sum(g1, src, dst, zerosH)
    g2 = _tc_mid(p1, g1, dinv, gcn_b1.reshape(1, H), gcn_W2)
    p2 = _sc_edge_sum(g2, src, dst, zerosH)

    return _tc_final(p2, g2, dinv, gcn_b2.reshape(1, H), pw2_W,
                     pw2_b.reshape(1, H), proj1)


# speedup vs baseline: 19.3244x; 1.9757x over previous
import functools

import jax
import jax.numpy as jnp
from jax import lax
from jax.experimental import pallas as pl
from jax.experimental.pallas import tpu as pltpu
from jax.experimental.pallas import tpu_sc as plsc

N_NODES = 10000
N_EDGES = 320000
D_IN = 771
H = 128
ALPHA = 0.1
FRAC = 0.5

NUM_CORES = 2
NUM_SUBCORES = 16
NUM_TILES = NUM_CORES * NUM_SUBCORES
ROWS_PER_SUBCORE = 624
ROWS_TAIL = N_NODES - NUM_SUBCORES * ROWS_PER_SUBCORE

CHUNK = 80
EDGES_PER_TILE = N_EDGES // NUM_TILES
NUM_CHUNKS = EDGES_PER_TILE // CHUNK

ECHUNK = 128
NUM_ECHUNKS = N_EDGES // ECHUNK

_MESH = plsc.VectorSubcoreMesh(core_axis_name="c", subcore_axis_name="s")


def _zero_acc(s, zeros_hbm, acc):
    row0 = s * ROWS_PER_SUBCORE
    tail0 = NUM_SUBCORES * ROWS_PER_SUBCORE
    pltpu.sync_copy(zeros_hbm.at[pl.ds(0, ROWS_PER_SUBCORE)],
                    acc.at[pl.ds(row0, ROWS_PER_SUBCORE)])

    @pl.when(s == NUM_SUBCORES - 1)
    def _():
        pltpu.sync_copy(zeros_hbm.at[pl.ds(0, ROWS_TAIL)],
                        acc.at[pl.ds(tail0, ROWS_TAIL)])


def _write_out(c, s, acc, out_hbm):
    row0 = s * ROWS_PER_SUBCORE
    tail0 = NUM_SUBCORES * ROWS_PER_SUBCORE
    pltpu.sync_copy(acc.at[pl.ds(row0, ROWS_PER_SUBCORE)],
                    out_hbm.at[c, pl.ds(row0, ROWS_PER_SUBCORE)])

    @pl.when(s == NUM_SUBCORES - 1)
    def _():
        pltpu.sync_copy(acc.at[pl.ds(tail0, ROWS_TAIL)],
                        out_hbm.at[c, pl.ds(tail0, ROWS_TAIL)])


@functools.partial(
    pl.kernel,
    mesh=_MESH,
    out_type=jax.ShapeDtypeStruct((NUM_CORES, N_NODES, H), jnp.float32),
    scratch_types=[
        pltpu.VMEM((CHUNK,), jnp.int32),
        pltpu.VMEM((CHUNK, H), jnp.float32),
        pltpu.VMEM_SHARED((N_NODES, H), jnp.float32),
    ],
)
def _sc_degree(dst_hbm, zeros_hbm, ones_hbm, out_hbm, didx, ones_v, acc):
    c = lax.axis_index("c")
    s = lax.axis_index("s")

    pltpu.sync_copy(ones_hbm, ones_v)
    _zero_acc(s, zeros_hbm, acc)
    plsc.subcore_barrier()

    tile_base = (c * NUM_SUBCORES + s) * EDGES_PER_TILE

    def body(i, _):
        base = tile_base + i * CHUNK
        pltpu.sync_copy(dst_hbm.at[pl.ds(base, CHUNK)], didx)
        pltpu.sync_copy(ones_v, acc.at[didx], add=True)
        return _
    lax.fori_loop(0, NUM_CHUNKS, body, None)

    plsc.subcore_barrier()
    _write_out(c, s, acc, out_hbm)


@functools.partial(
    pl.kernel,
    mesh=_MESH,
    out_type=jax.ShapeDtypeStruct((NUM_CORES, N_NODES, H), jnp.float32),
    scratch_types=[
        pltpu.VMEM((2, ECHUNK), jnp.int32),
        pltpu.VMEM((2, ECHUNK), jnp.int32),
        pltpu.VMEM((ECHUNK, H), jnp.float32),
        pltpu.VMEM((ECHUNK, H), jnp.float32),
        pltpu.VMEM_SHARED((N_NODES, H), jnp.float32),
        pltpu.SemaphoreType.DMA,
        pltpu.SemaphoreType.DMA,
        pltpu.SemaphoreType.DMA,
        pltpu.SemaphoreType.DMA,
    ],
)
def _sc_edge_sum(g_hbm, edge_hbm, zeros_hbm, out_hbm,
                 idx0, idx1, rows0, rows1, acc, isem0, isem1, gsem0, gsem1):
    c = lax.axis_index("c")
    s = lax.axis_index("s")
    t = c * NUM_SUBCORES + s

    _zero_acc(s, zeros_hbm, acc)

    lo = (t * NUM_ECHUNKS) // NUM_TILES
    hi = ((t + 1) * NUM_ECHUNKS) // NUM_TILES
    n = hi - lo

    idx = (idx0, idx1)
    rows = (rows0, rows1)
    isem = (isem0, isem1)
    gsem = (gsem0, gsem1)

    def idx_src(j):
        return edge_hbm.at[:, pl.ds((lo + j) * ECHUNK, ECHUNK)]

    pltpu.sync_copy(idx_src(0), idx0)
    pltpu.async_copy(g_hbm.at[idx0.at[0]], rows0, gsem0)

    @pl.when(1 < n)
    def _():
        pltpu.async_copy(idx_src(1), idx1, isem1)

    plsc.subcore_barrier()

    def body(j, _):
        for p in range(2):
            @pl.when(j % 2 == p)
            def _():
                ib, rb = idx[p], rows[p]
                ob, orb = idx[1 - p], rows[1 - p]

                @pl.when(j + 1 < n)
                def _():
                    pltpu.make_async_copy(idx_src(j + 1), ob, isem[1 - p]).wait()
                    pltpu.async_copy(g_hbm.at[ob.at[0]], orb, gsem[1 - p])

                pltpu.make_async_copy(g_hbm.at[ib.at[0]], rb, gsem[p]).wait()
                pltpu.sync_copy(rb, acc.at[ib.at[1]], add=True)

                @pl.when(j + 2 < n)
                def _():
                    pltpu.async_copy(idx_src(j + 2), ib, isem[p])
        return _

    lax.fori_loop(0, n, body, None)

    plsc.subcore_barrier()
    _write_out(c, s, acc, out_hbm)


_R = 1000


def _tc_front_body(x_ref, w0_ref, mlpw_ref, mlpb_ref, pw1w_ref, pw1b_ref,
                   degp_ref, g0_ref, proj1_ref, dinv_ref):
    x = x_ref[...]
    deg = degp_ref[0, :, 0] + degp_ref[1, :, 0] + 1.0
    dinv = lax.rsqrt(jnp.maximum(deg, 1.0))[:, None]
    dinv_ref[...] = dinv
    g0_ref[...] = jnp.dot(x, w0_ref[...], preferred_element_type=jnp.float32) * dinv
    r1 = jnp.maximum(
        jnp.dot(x, mlpw_ref[...], preferred_element_type=jnp.float32)
        + mlpb_ref[...], 0.0)
    t = jnp.dot(r1, pw1w_ref[...], preferred_element_type=jnp.float32) + pw1b_ref[...]
    proj1_ref[...] = jnp.where(t > 0, t, ALPHA * t)


def _tc_mid_body(p_ref, g_ref, dinv_ref, b_ref, w_ref, gout_ref):
    dinv = dinv_ref[...]
    h = jnp.maximum((p_ref[0] + p_ref[1] + g_ref[...]) * dinv + b_ref[...], 0.0)
    gout_ref[...] = jnp.dot(h, w_ref[...], preferred_element_type=jnp.float32) * dinv


def _tc_final_body(p_ref, g_ref, dinv_ref, b_ref, pw2w_ref, pw2b_ref,
                   proj1_ref, out_ref):
    dinv = dinv_ref[...]
    rna2d = jnp.maximum((p_ref[0] + p_ref[1] + g_ref[...]) * dinv + b_ref[...], 0.0)
    t = jnp.dot(rna2d, pw2w_ref[...], preferred_element_type=jnp.float32) + pw2b_ref[...]
    proj2 = jnp.where(t > 0, t, ALPHA * t)
    out_ref[...] = FRAC * proj2 + (1.0 - FRAC) * proj1_ref[...]


def _row_spec(width):
    return pl.BlockSpec((_R, width), lambda i: (i, 0))


def _full_spec(shape):
    nd = len(shape)
    return pl.BlockSpec(shape, lambda i: (0,) * nd)


def _part_spec(width):
    return pl.BlockSpec((NUM_CORES, _R, width), lambda i: (0, i, 0))


_GRID = (N_NODES // _R,)


def _tc_front(x, w0, mlpw, mlpb, pw1w, pw1b, degp):
    return pl.pallas_call(
        _tc_front_body,
        grid=_GRID,
        in_specs=[
            _row_spec(D_IN),
            _full_spec((D_IN, H)),
            _full_spec((D_IN, H)),
            _full_spec((1, H)),
            _full_spec((H, H)),
            _full_spec((1, H)),
            _part_spec(H),
        ],
        out_specs=[_row_spec(H), _row_spec(H), _row_spec(1)],
        out_shape=[
            jax.ShapeDtypeStruct((N_NODES, H), jnp.float32),
            jax.ShapeDtypeStruct((N_NODES, H), jnp.float32),
            jax.ShapeDtypeStruct((N_NODES, 1), jnp.float32),
        ],
    )(x, w0, mlpw, mlpb, pw1w, pw1b, degp)


def _tc_mid(p, g, dinv, b, w):
    return pl.pallas_call(
        _tc_mid_body,
        grid=_GRID,
        in_specs=[
            _part_spec(H),
            _row_spec(H),
            _row_spec(1),
            _full_spec((1, H)),
            _full_spec((H, H)),
        ],
        out_specs=_row_spec(H),
        out_shape=jax.ShapeDtypeStruct((N_NODES, H), jnp.float32),
    )(p, g, dinv, b, w)


def _tc_final(p, g, dinv, b, pw2w, pw2b, proj1):
    return pl.pallas_call(
        _tc_final_body,
        grid=_GRID,
        in_specs=[
            _part_spec(H),
            _row_spec(H),
            _row_spec(1),
            _full_spec((1, H)),
            _full_spec((H, H)),
            _full_spec((1, H)),
            _row_spec(H),
        ],
        out_specs=_row_spec(H),
        out_shape=jax.ShapeDtypeStruct((N_NODES, H), jnp.float32),
    )(p, g, dinv, b, pw2w, pw2b, proj1)


def kernel(x, gcn_W0, gcn_b0, gcn_W1, gcn_b1, gcn_W2, gcn_b2,
           mlp_W, mlp_b, pw2_W, pw2_b, pw1_W, pw1_b, edge_index):
    edge = edge_index.astype(jnp.int32)
    dst = edge[1]

    zerosH = jnp.zeros((ROWS_PER_SUBCORE, H), jnp.float32)
    onesH = jnp.ones((CHUNK, H), jnp.float32)

    degp = _sc_degree(dst, zerosH, onesH)

    g0, proj1, dinv = _tc_front(
        x, gcn_W0, mlp_W, mlp_b.reshape(1, H), pw1_W, pw1_b.reshape(1, H), degp)

    p0 = _sc_edge_sum(g0, edge, zerosH)
    g1 = _tc_mid(p0, g0, dinv, gcn_b0.reshape(1, H), gcn_W1)
    p1 = _sc_edge_sum(g1, edge, zerosH)
    g2 = _tc_mid(p1, g1, dinv, gcn_b1.reshape(1, H), gcn_W2)
    p2 = _sc_edge_sum(g2, edge, zerosH)

    return _tc_final(p2, g2, dinv, gcn_b2.reshape(1, H), pw2_W,
                     pw2_b.reshape(1, H), proj1)


# speedup vs baseline: 21.5256x; 1.1139x over previous
import functools

import jax
import jax.numpy as jnp
from jax import lax
from jax.experimental import pallas as pl
from jax.experimental.pallas import tpu as pltpu
from jax.experimental.pallas import tpu_sc as plsc

N_NODES = 10000
N_EDGES = 320000
D_IN = 771
H = 128
ALPHA = 0.1
FRAC = 0.5

NUM_CORES = 2
NUM_SUBCORES = 16
NUM_TILES = NUM_CORES * NUM_SUBCORES
ROWS_PER_SUBCORE = 624
ROWS_TAIL = N_NODES - NUM_SUBCORES * ROWS_PER_SUBCORE

CHUNK = 80
EDGES_PER_TILE = N_EDGES // NUM_TILES
NUM_CHUNKS = EDGES_PER_TILE // CHUNK

ECHUNK = 128
NUM_ECHUNKS = N_EDGES // ECHUNK

_MESH = plsc.VectorSubcoreMesh(core_axis_name="c", subcore_axis_name="s")


def _zero_acc(s, zeros_hbm, acc):
    row0 = s * ROWS_PER_SUBCORE
    tail0 = NUM_SUBCORES * ROWS_PER_SUBCORE
    pltpu.sync_copy(zeros_hbm.at[pl.ds(0, ROWS_PER_SUBCORE)],
                    acc.at[pl.ds(row0, ROWS_PER_SUBCORE)])

    @pl.when(s == NUM_SUBCORES - 1)
    def _():
        pltpu.sync_copy(zeros_hbm.at[pl.ds(0, ROWS_TAIL)],
                        acc.at[pl.ds(tail0, ROWS_TAIL)])


def _write_out(c, s, acc, out_hbm):
    row0 = s * ROWS_PER_SUBCORE
    tail0 = NUM_SUBCORES * ROWS_PER_SUBCORE
    pltpu.sync_copy(acc.at[pl.ds(row0, ROWS_PER_SUBCORE)],
                    out_hbm.at[c, pl.ds(row0, ROWS_PER_SUBCORE)])

    @pl.when(s == NUM_SUBCORES - 1)
    def _():
        pltpu.sync_copy(acc.at[pl.ds(tail0, ROWS_TAIL)],
                        out_hbm.at[c, pl.ds(tail0, ROWS_TAIL)])


@functools.partial(
    pl.kernel,
    mesh=_MESH,
    out_type=jax.ShapeDtypeStruct((NUM_CORES, N_NODES, H), jnp.float32),
    scratch_types=[
        pltpu.VMEM((ECHUNK,), jnp.int32),
        pltpu.VMEM((ECHUNK,), jnp.int32),
        pltpu.VMEM((ECHUNK, H), jnp.float32),
        pltpu.VMEM_SHARED((N_NODES, H), jnp.float32),
        pltpu.SemaphoreType.DMA,
        pltpu.SemaphoreType.DMA,
    ],
)
def _sc_degree(dst_hbm, zeros_hbm, ones_hbm, out_hbm, idx0, idx1, ones_v, acc,
               isem0, isem1):
    c = lax.axis_index("c")
    s = lax.axis_index("s")
    t = c * NUM_SUBCORES + s

    pltpu.sync_copy(ones_hbm, ones_v)
    _zero_acc(s, zeros_hbm, acc)

    lo = (t * NUM_ECHUNKS) // NUM_TILES
    hi = ((t + 1) * NUM_ECHUNKS) // NUM_TILES
    n = hi - lo

    idx = (idx0, idx1)
    isem = (isem0, isem1)

    def isrc(j):
        return dst_hbm.at[pl.ds((lo + j) * ECHUNK, ECHUNK)]

    pltpu.sync_copy(isrc(0), idx0)
    plsc.subcore_barrier()

    def body(j, _):
        for p in range(2):
            @pl.when(j % 2 == p)
            def _():
                @pl.when(j + 1 < n)
                def _():
                    pltpu.async_copy(isrc(j + 1), idx[1 - p], isem[1 - p])

                pltpu.sync_copy(ones_v, acc.at[idx[p]], add=True)

                @pl.when(j + 1 < n)
                def _():
                    pltpu.make_async_copy(isrc(j + 1), idx[1 - p],
                                          isem[1 - p]).wait()
        return _
    lax.fori_loop(0, n, body, None)

    plsc.subcore_barrier()
    _write_out(c, s, acc, out_hbm)


@functools.partial(
    pl.kernel,
    mesh=_MESH,
    out_type=jax.ShapeDtypeStruct((NUM_CORES, N_NODES, H), jnp.float32),
    scratch_types=[
        pltpu.VMEM((2, ECHUNK), jnp.int32),
        pltpu.VMEM((2, ECHUNK), jnp.int32),
        pltpu.VMEM((ECHUNK, H), jnp.float32),
        pltpu.VMEM((ECHUNK, H), jnp.float32),
        pltpu.VMEM_SHARED((N_NODES, H), jnp.float32),
        pltpu.SemaphoreType.DMA,
        pltpu.SemaphoreType.DMA,
        pltpu.SemaphoreType.DMA,
        pltpu.SemaphoreType.DMA,
    ],
)
def _sc_edge_sum(g_hbm, edge_hbm, zeros_hbm, out_hbm,
                 idx0, idx1, rows0, rows1, acc, isem0, isem1, gsem0, gsem1):
    c = lax.axis_index("c")
    s = lax.axis_index("s")
    t = c * NUM_SUBCORES + s

    _zero_acc(s, zeros_hbm, acc)

    lo = (t * NUM_ECHUNKS) // NUM_TILES
    hi = ((t + 1) * NUM_ECHUNKS) // NUM_TILES
    n = hi - lo

    idx = (idx0, idx1)
    rows = (rows0, rows1)
    isem = (isem0, isem1)
    gsem = (gsem0, gsem1)

    def idx_src(j):
        return edge_hbm.at[:, pl.ds((lo + j) * ECHUNK, ECHUNK)]

    pltpu.sync_copy(idx_src(0), idx0)
    pltpu.async_copy(g_hbm.at[idx0.at[0]], rows0, gsem0)

    @pl.when(1 < n)
    def _():
        pltpu.async_copy(idx_src(1), idx1, isem1)

    plsc.subcore_barrier()

    def body(j, _):
        for p in range(2):
            @pl.when(j % 2 == p)
            def _():
                ib, rb = idx[p], rows[p]
                ob, orb = idx[1 - p], rows[1 - p]

                @pl.when(j + 1 < n)
                def _():
                    pltpu.make_async_copy(idx_src(j + 1), ob, isem[1 - p]).wait()
                    pltpu.async_copy(g_hbm.at[ob.at[0]], orb, gsem[1 - p])

                pltpu.make_async_copy(g_hbm.at[ib.at[0]], rb, gsem[p]).wait()
                pltpu.sync_copy(rb, acc.at[ib.at[1]], add=True)

                @pl.when(j + 2 < n)
                def _():
                    pltpu.async_copy(idx_src(j + 2), ib, isem[p])
        return _

    lax.fori_loop(0, n, body, None)

    plsc.subcore_barrier()
    _write_out(c, s, acc, out_hbm)


_R = 1000


def _tc_dense_body(x_ref, w0_ref, mlpw_ref, mlpb_ref, pw1w_ref, pw1b_ref,
                   h0_ref, proj1_ref):
    x = x_ref[...]
    h0_ref[...] = jnp.dot(x, w0_ref[...], preferred_element_type=jnp.float32)
    r1 = jnp.maximum(
        jnp.dot(x, mlpw_ref[...], preferred_element_type=jnp.float32)
        + mlpb_ref[...], 0.0)
    t = jnp.dot(r1, pw1w_ref[...], preferred_element_type=jnp.float32) + pw1b_ref[...]
    proj1_ref[...] = jnp.where(t > 0, t, ALPHA * t)


def _tc_scale_body(degp_ref, h0_ref, g0_ref, dinv_ref):
    deg = degp_ref[0, :, 0] + degp_ref[1, :, 0] + 1.0
    dinv = lax.rsqrt(jnp.maximum(deg, 1.0))[:, None]
    dinv_ref[...] = dinv
    g0_ref[...] = h0_ref[...] * dinv


def _tc_mid_body(p_ref, g_ref, dinv_ref, b_ref, w_ref, gout_ref):
    dinv = dinv_ref[...]
    h = jnp.maximum((p_ref[0] + p_ref[1] + g_ref[...]) * dinv + b_ref[...], 0.0)
    gout_ref[...] = jnp.dot(h, w_ref[...], preferred_element_type=jnp.float32) * dinv


def _tc_final_body(p_ref, g_ref, dinv_ref, b_ref, pw2w_ref, pw2b_ref,
                   proj1_ref, out_ref):
    dinv = dinv_ref[...]
    rna2d = jnp.maximum((p_ref[0] + p_ref[1] + g_ref[...]) * dinv + b_ref[...], 0.0)
    t = jnp.dot(rna2d, pw2w_ref[...], preferred_element_type=jnp.float32) + pw2b_ref[...]
    proj2 = jnp.where(t > 0, t, ALPHA * t)
    out_ref[...] = FRAC * proj2 + (1.0 - FRAC) * proj1_ref[...]


def _row_spec(width):
    return pl.BlockSpec((_R, width), lambda i: (i, 0))


def _full_spec(shape):
    nd = len(shape)
    return pl.BlockSpec(shape, lambda i: (0,) * nd)


def _part_spec(width):
    return pl.BlockSpec((NUM_CORES, _R, width), lambda i: (0, i, 0))


_GRID = (N_NODES // _R,)


def _tc_dense(x, w0, mlpw, mlpb, pw1w, pw1b):
    return pl.pallas_call(
        _tc_dense_body,
        grid=_GRID,
        in_specs=[
            _row_spec(D_IN),
            _full_spec((D_IN, H)),
            _full_spec((D_IN, H)),
            _full_spec((1, H)),
            _full_spec((H, H)),
            _full_spec((1, H)),
        ],
        out_specs=[_row_spec(H), _row_spec(H)],
        out_shape=[
            jax.ShapeDtypeStruct((N_NODES, H), jnp.float32),
            jax.ShapeDtypeStruct((N_NODES, H), jnp.float32),
        ],
    )(x, w0, mlpw, mlpb, pw1w, pw1b)


def _tc_scale(degp, h0):
    return pl.pallas_call(
        _tc_scale_body,
        grid=_GRID,
        in_specs=[_part_spec(H), _row_spec(H)],
        out_specs=[_row_spec(H), _row_spec(1)],
        out_shape=[
            jax.ShapeDtypeStruct((N_NODES, H), jnp.float32),
            jax.ShapeDtypeStruct((N_NODES, 1), jnp.float32),
        ],
    )(degp, h0)


def _tc_mid(p, g, dinv, b, w):
    return pl.pallas_call(
        _tc_mid_body,
        grid=_GRID,
        in_specs=[
            _part_spec(H),
            _row_spec(H),
            _row_spec(1),
            _full_spec((1, H)),
            _full_spec((H, H)),
        ],
        out_specs=_row_spec(H),
        out_shape=jax.ShapeDtypeStruct((N_NODES, H), jnp.float32),
    )(p, g, dinv, b, w)


def _tc_final(p, g, dinv, b, pw2w, pw2b, proj1):
    return pl.pallas_call(
        _tc_final_body,
        grid=_GRID,
        in_specs=[
            _part_spec(H),
            _row_spec(H),
            _row_spec(1),
            _full_spec((1, H)),
            _full_spec((H, H)),
            _full_spec((1, H)),
            _row_spec(H),
        ],
        out_specs=_row_spec(H),
        out_shape=jax.ShapeDtypeStruct((N_NODES, H), jnp.float32),
    )(p, g, dinv, b, pw2w, pw2b, proj1)


def kernel(x, gcn_W0, gcn_b0, gcn_W1, gcn_b1, gcn_W2, gcn_b2,
           mlp_W, mlp_b, pw2_W, pw2_b, pw1_W, pw1_b, edge_index):
    edge = edge_index.astype(jnp.int32)
    dst = edge[1]

    zerosH = jnp.zeros((ROWS_PER_SUBCORE, H), jnp.float32)
    onesH = jnp.ones((ECHUNK, H), jnp.float32)

    degp = _sc_degree(dst, zerosH, onesH)
    h0, proj1 = _tc_dense(
        x, gcn_W0, mlp_W, mlp_b.reshape(1, H), pw1_W, pw1_b.reshape(1, H))
    g0, dinv = _tc_scale(degp, h0)

    p0 = _sc_edge_sum(g0, edge, zerosH)
    g1 = _tc_mid(p0, g0, dinv, gcn_b0.reshape(1, H), gcn_W1)
    p1 = _sc_edge_sum(g1, edge, zerosH)
    g2 = _tc_mid(p1, g1, dinv, gcn_b1.reshape(1, H), gcn_W2)
    p2 = _sc_edge_sum(g2, edge, zerosH)

    return _tc_final(p2, g2, dinv, gcn_b2.reshape(1, H), pw2_W,
                     pw2_b.reshape(1, H), proj1)


# speedup vs baseline: 23.5696x; 1.0950x over previous
import functools

import jax
import jax.numpy as jnp
from jax import lax
from jax.experimental import pallas as pl
from jax.experimental.pallas import tpu as pltpu
from jax.experimental.pallas import tpu_sc as plsc

N_NODES = 10000
N_EDGES = 320000
D_IN = 771
H = 128
ALPHA = 0.1
FRAC = 0.5

NUM_CORES = 2
NUM_SUBCORES = 16
NUM_TILES = NUM_CORES * NUM_SUBCORES
ROWS_PER_SUBCORE = 624
ROWS_TAIL = N_NODES - NUM_SUBCORES * ROWS_PER_SUBCORE

CHUNK = 80
EDGES_PER_TILE = N_EDGES // NUM_TILES
NUM_CHUNKS = EDGES_PER_TILE // CHUNK

ECHUNK = 128
NUM_ECHUNKS = N_EDGES // ECHUNK

_MESH = plsc.VectorSubcoreMesh(core_axis_name="c", subcore_axis_name="s")


def _zero_acc(s, zeros_hbm, acc):
    row0 = s * ROWS_PER_SUBCORE
    tail0 = NUM_SUBCORES * ROWS_PER_SUBCORE
    pltpu.sync_copy(zeros_hbm.at[pl.ds(0, ROWS_PER_SUBCORE)],
                    acc.at[pl.ds(row0, ROWS_PER_SUBCORE)])

    @pl.when(s == NUM_SUBCORES - 1)
    def _():
        pltpu.sync_copy(zeros_hbm.at[pl.ds(0, ROWS_TAIL)],
                        acc.at[pl.ds(tail0, ROWS_TAIL)])


def _write_out(c, s, acc, out_hbm):
    row0 = s * ROWS_PER_SUBCORE
    tail0 = NUM_SUBCORES * ROWS_PER_SUBCORE
    pltpu.sync_copy(acc.at[pl.ds(row0, ROWS_PER_SUBCORE)],
                    out_hbm.at[c, pl.ds(row0, ROWS_PER_SUBCORE)])

    @pl.when(s == NUM_SUBCORES - 1)
    def _():
        pltpu.sync_copy(acc.at[pl.ds(tail0, ROWS_TAIL)],
                        out_hbm.at[c, pl.ds(tail0, ROWS_TAIL)])


@functools.partial(
    pl.kernel,
    mesh=_MESH,
    out_type=jax.ShapeDtypeStruct((NUM_CORES, N_NODES, H), jnp.float32),
    scratch_types=[
        pltpu.VMEM((ECHUNK,), jnp.int32),
        pltpu.VMEM((ECHUNK,), jnp.int32),
        pltpu.VMEM((ECHUNK, H), jnp.float32),
        pltpu.VMEM_SHARED((N_NODES, H), jnp.float32),
        pltpu.SemaphoreType.DMA,
        pltpu.SemaphoreType.DMA,
    ],
)
def _sc_degree(dst_hbm, zeros_hbm, ones_hbm, out_hbm, idx0, idx1, ones_v, acc,
               isem0, isem1):
    c = lax.axis_index("c")
    s = lax.axis_index("s")
    t = c * NUM_SUBCORES + s

    pltpu.sync_copy(ones_hbm, ones_v)
    _zero_acc(s, zeros_hbm, acc)

    lo = (t * NUM_ECHUNKS) // NUM_TILES
    hi = ((t + 1) * NUM_ECHUNKS) // NUM_TILES
    n = hi - lo

    idx = (idx0, idx1)
    isem = (isem0, isem1)

    def isrc(j):
        return dst_hbm.at[pl.ds((lo + j) * ECHUNK, ECHUNK)]

    pltpu.sync_copy(isrc(0), idx0)
    plsc.subcore_barrier()

    def body(j, _):
        for p in range(2):
            @pl.when(j % 2 == p)
            def _():
                @pl.when(j + 1 < n)
                def _():
                    pltpu.async_copy(isrc(j + 1), idx[1 - p], isem[1 - p])

                pltpu.sync_copy(ones_v, acc.at[idx[p]], add=True)

                @pl.when(j + 1 < n)
                def _():
                    pltpu.make_async_copy(isrc(j + 1), idx[1 - p],
                                          isem[1 - p]).wait()
        return _
    lax.fori_loop(0, n, body, None)

    plsc.subcore_barrier()
    _write_out(c, s, acc, out_hbm)


@functools.partial(
    pl.kernel,
    mesh=_MESH,
    out_type=jax.ShapeDtypeStruct((NUM_CORES, N_NODES, H), jnp.float32),
    scratch_types=[
        pltpu.VMEM((2, ECHUNK), jnp.int32),
        pltpu.VMEM((2, ECHUNK), jnp.int32),
        pltpu.VMEM((2, ECHUNK), jnp.int32),
        pltpu.VMEM((ECHUNK, H), jnp.float32),
        pltpu.VMEM((ECHUNK, H), jnp.float32),
        pltpu.VMEM_SHARED((N_NODES, H), jnp.float32),
        pltpu.SemaphoreType.DMA,
        pltpu.SemaphoreType.DMA,
        pltpu.SemaphoreType.DMA,
        pltpu.SemaphoreType.DMA,
        pltpu.SemaphoreType.DMA,
        pltpu.SemaphoreType.DMA,
        pltpu.SemaphoreType.DMA,
    ],
)
def _sc_edge_sum(g_hbm, edge_hbm, zeros_hbm, out_hbm,
                 idx0, idx1, idx2, rows0, rows1, acc,
                 isem0, isem1, isem2, gsem0, gsem1, ssem0, ssem1):
    c = lax.axis_index("c")
    s = lax.axis_index("s")
    t = c * NUM_SUBCORES + s

    _zero_acc(s, zeros_hbm, acc)

    lo = (t * NUM_ECHUNKS) // NUM_TILES
    hi = ((t + 1) * NUM_ECHUNKS) // NUM_TILES
    n = hi - lo

    idx = (idx0, idx1, idx2)
    rows = (rows0, rows1)
    isem = (isem0, isem1, isem2)
    gsem = (gsem0, gsem1)
    ssem = (ssem0, ssem1)

    def idx_src(j):
        return edge_hbm.at[:, pl.ds((lo + j) * ECHUNK, ECHUNK)]

    pltpu.sync_copy(idx_src(0), idx0)
    pltpu.async_copy(g_hbm.at[idx0.at[0]], rows0, gsem0)

    @pl.when(1 < n)
    def _():
        pltpu.async_copy(idx_src(1), idx1, isem1)

    plsc.subcore_barrier()

    def body(j, _):
        for p2 in range(2):
            for p3 in range(3):
                @pl.when(jnp.logical_and(j % 2 == p2, j % 3 == p3))
                def _():
                    @pl.when(j >= 1)
                    def _():
                        pltpu.make_async_copy(
                            rows[1 - p2], acc.at[idx[(p3 + 2) % 3].at[1]],
                            ssem[1 - p2]).wait()

                    @pl.when(j + 1 < n)
                    def _():
                        pltpu.make_async_copy(
                            idx_src(j + 1), idx[(p3 + 1) % 3],
                            isem[(p3 + 1) % 3]).wait()
                        pltpu.async_copy(
                            g_hbm.at[idx[(p3 + 1) % 3].at[0]],
                            rows[1 - p2], gsem[1 - p2])

                    pltpu.make_async_copy(
                        g_hbm.at[idx[p3].at[0]], rows[p2], gsem[p2]).wait()
                    pltpu.async_copy(
                        rows[p2], acc.at[idx[p3].at[1]], ssem[p2], add=True)

                    @pl.when(j + 2 < n)
                    def _():
                        pltpu.async_copy(idx_src(j + 2), idx[(p3 + 2) % 3],
                                         isem[(p3 + 2) % 3])
        return _

    lax.fori_loop(0, n, body, None)

    for p2 in range(2):
        for p3 in range(3):
            @pl.when(jnp.logical_and((n - 1) % 2 == p2, (n - 1) % 3 == p3))
            def _():
                pltpu.make_async_copy(
                    rows[p2], acc.at[idx[p3].at[1]], ssem[p2]).wait()

    plsc.subcore_barrier()
    _write_out(c, s, acc, out_hbm)


_R = 1000


def _tc_dense_body(x_ref, w0_ref, mlpw_ref, mlpb_ref, pw1w_ref, pw1b_ref,
                   h0_ref, proj1_ref):
    x = x_ref[...]
    h0_ref[...] = jnp.dot(x, w0_ref[...], preferred_element_type=jnp.float32)
    r1 = jnp.maximum(
        jnp.dot(x, mlpw_ref[...], preferred_element_type=jnp.float32)
        + mlpb_ref[...], 0.0)
    t = jnp.dot(r1, pw1w_ref[...], preferred_element_type=jnp.float32) + pw1b_ref[...]
    proj1_ref[...] = jnp.where(t > 0, t, ALPHA * t)


def _tc_scale_body(degp_ref, h0_ref, g0_ref, dinv_ref):
    deg = degp_ref[0, :, 0] + degp_ref[1, :, 0] + 1.0
    dinv = lax.rsqrt(jnp.maximum(deg, 1.0))[:, None]
    dinv_ref[...] = dinv
    g0_ref[...] = h0_ref[...] * dinv


def _tc_mid_body(p_ref, g_ref, dinv_ref, b_ref, w_ref, gout_ref):
    dinv = dinv_ref[...]
    h = jnp.maximum((p_ref[0] + p_ref[1] + g_ref[...]) * dinv + b_ref[...], 0.0)
    gout_ref[...] = jnp.dot(h, w_ref[...], preferred_element_type=jnp.float32) * dinv


def _tc_final_body(p_ref, g_ref, dinv_ref, b_ref, pw2w_ref, pw2b_ref,
                   proj1_ref, out_ref):
    dinv = dinv_ref[...]
    rna2d = jnp.maximum((p_ref[0] + p_ref[1] + g_ref[...]) * dinv + b_ref[...], 0.0)
    t = jnp.dot(rna2d, pw2w_ref[...], preferred_element_type=jnp.float32) + pw2b_ref[...]
    proj2 = jnp.where(t > 0, t, ALPHA * t)
    out_ref[...] = FRAC * proj2 + (1.0 - FRAC) * proj1_ref[...]


def _row_spec(width):
    return pl.BlockSpec((_R, width), lambda i: (i, 0))


def _full_spec(shape):
    nd = len(shape)
    return pl.BlockSpec(shape, lambda i: (0,) * nd)


def _part_spec(width):
    return pl.BlockSpec((NUM_CORES, _R, width), lambda i: (0, i, 0))


_GRID = (N_NODES // _R,)


def _tc_dense(x, w0, mlpw, mlpb, pw1w, pw1b):
    return pl.pallas_call(
        _tc_dense_body,
        grid=_GRID,
        in_specs=[
            _row_spec(D_IN),
            _full_spec((D_IN, H)),
            _full_spec((D_IN, H)),
            _full_spec((1, H)),
            _full_spec((H, H)),
            _full_spec((1, H)),
        ],
        out_specs=[_row_spec(H), _row_spec(H)],
        out_shape=[
            jax.ShapeDtypeStruct((N_NODES, H), jnp.float32),
            jax.ShapeDtypeStruct((N_NODES, H), jnp.float32),
        ],
    )(x, w0, mlpw, mlpb, pw1w, pw1b)


def _tc_scale(degp, h0):
    return pl.pallas_call(
        _tc_scale_body,
        grid=_GRID,
        in_specs=[_part_spec(H), _row_spec(H)],
        out_specs=[_row_spec(H), _row_spec(1)],
        out_shape=[
            jax.ShapeDtypeStruct((N_NODES, H), jnp.float32),
            jax.ShapeDtypeStruct((N_NODES, 1), jnp.float32),
        ],
    )(degp, h0)


def _tc_mid(p, g, dinv, b, w):
    return pl.pallas_call(
        _tc_mid_body,
        grid=_GRID,
        in_specs=[
            _part_spec(H),
            _row_spec(H),
            _row_spec(1),
            _full_spec((1, H)),
            _full_spec((H, H)),
        ],
        out_specs=_row_spec(H),
        out_shape=jax.ShapeDtypeStruct((N_NODES, H), jnp.float32),
    )(p, g, dinv, b, w)


def _tc_final(p, g, dinv, b, pw2w, pw2b, proj1):
    return pl.pallas_call(
        _tc_final_body,
        grid=_GRID,
        in_specs=[
            _part_spec(H),
            _row_spec(H),
            _row_spec(1),
            _full_spec((1, H)),
            _full_spec((H, H)),
            _full_spec((1, H)),
            _row_spec(H),
        ],
        out_specs=_row_spec(H),
        out_shape=jax.ShapeDtypeStruct((N_NODES, H), jnp.float32),
    )(p, g, dinv, b, pw2w, pw2b, proj1)


def kernel(x, gcn_W0, gcn_b0, gcn_W1, gcn_b1, gcn_W2, gcn_b2,
           mlp_W, mlp_b, pw2_W, pw2_b, pw1_W, pw1_b, edge_index):
    edge = edge_index.astype(jnp.int32)
    dst = edge[1]

    zerosH = jnp.zeros((ROWS_PER_SUBCORE, H), jnp.float32)
    onesH = jnp.ones((ECHUNK, H), jnp.float32)

    degp = _sc_degree(dst, zerosH, onesH)
    h0, proj1 = _tc_dense(
        x, gcn_W0, mlp_W, mlp_b.reshape(1, H), pw1_W, pw1_b.reshape(1, H))
    g0, dinv = _tc_scale(degp, h0)

    p0 = _sc_edge_sum(g0, edge, zerosH)
    g1 = _tc_mid(p0, g0, dinv, gcn_b0.reshape(1, H), gcn_W1)
    p1 = _sc_edge_sum(g1, edge, zerosH)
    g2 = _tc_mid(p1, g1, dinv, gcn_b1.reshape(1, H), gcn_W2)
    p2 = _sc_edge_sum(g2, edge, zerosH)

    return _tc_final(p2, g2, dinv, gcn_b2.reshape(1, H), pw2_W,
                     pw2_b.reshape(1, H), proj1)


# speedup vs baseline: 23.6640x; 1.0040x over previous
import functools

import jax
import jax.numpy as jnp
from jax import lax
from jax.experimental import pallas as pl
from jax.experimental.pallas import tpu as pltpu
from jax.experimental.pallas import tpu_sc as plsc

N_NODES = 10000
N_EDGES = 320000
D_IN = 771
H = 128
ALPHA = 0.1
FRAC = 0.5

NUM_CORES = 2
NUM_SUBCORES = 16
NUM_TILES = NUM_CORES * NUM_SUBCORES
ROWS_PER_SUBCORE = 624
ROWS_TAIL = N_NODES - NUM_SUBCORES * ROWS_PER_SUBCORE

CHUNK = 80
EDGES_PER_TILE = N_EDGES // NUM_TILES
NUM_CHUNKS = EDGES_PER_TILE // CHUNK

ECHUNK = 128
NUM_ECHUNKS = N_EDGES // ECHUNK

_MESH = plsc.VectorSubcoreMesh(core_axis_name="c", subcore_axis_name="s")


def _zero_acc(s, zeros_hbm, acc):
    row0 = s * ROWS_PER_SUBCORE
    tail0 = NUM_SUBCORES * ROWS_PER_SUBCORE
    pltpu.sync_copy(zeros_hbm.at[pl.ds(0, ROWS_PER_SUBCORE)],
                    acc.at[pl.ds(row0, ROWS_PER_SUBCORE)])

    @pl.when(s == NUM_SUBCORES - 1)
    def _():
        pltpu.sync_copy(zeros_hbm.at[pl.ds(0, ROWS_TAIL)],
                        acc.at[pl.ds(tail0, ROWS_TAIL)])


def _write_out(c, s, acc, out_hbm):
    row0 = s * ROWS_PER_SUBCORE
    tail0 = NUM_SUBCORES * ROWS_PER_SUBCORE
    pltpu.sync_copy(acc.at[pl.ds(row0, ROWS_PER_SUBCORE)],
                    out_hbm.at[c, pl.ds(row0, ROWS_PER_SUBCORE)])

    @pl.when(s == NUM_SUBCORES - 1)
    def _():
        pltpu.sync_copy(acc.at[pl.ds(tail0, ROWS_TAIL)],
                        out_hbm.at[c, pl.ds(tail0, ROWS_TAIL)])


@functools.partial(
    pl.kernel,
    mesh=_MESH,
    out_type=jax.ShapeDtypeStruct((NUM_CORES, N_NODES, H), jnp.float32),
    scratch_types=[
        pltpu.VMEM((ECHUNK,), jnp.int32),
        pltpu.VMEM((ECHUNK,), jnp.int32),
        pltpu.VMEM((ECHUNK,), jnp.int32),
        pltpu.VMEM((ECHUNK, H), jnp.float32),
        pltpu.VMEM_SHARED((N_NODES, H), jnp.float32),
        pltpu.SemaphoreType.DMA,
        pltpu.SemaphoreType.DMA,
        pltpu.SemaphoreType.DMA,
        pltpu.SemaphoreType.DMA,
        pltpu.SemaphoreType.DMA,
    ],
)
def _sc_degree(dst_hbm, zeros_hbm, ones_hbm, out_hbm, idx0, idx1, idx2,
               ones_v, acc, isem0, isem1, isem2, ssem0, ssem1):
    c = lax.axis_index("c")
    s = lax.axis_index("s")
    t = c * NUM_SUBCORES + s

    pltpu.sync_copy(ones_hbm, ones_v)
    _zero_acc(s, zeros_hbm, acc)

    lo = (t * NUM_ECHUNKS) // NUM_TILES
    hi = ((t + 1) * NUM_ECHUNKS) // NUM_TILES
    n = hi - lo

    idx = (idx0, idx1, idx2)
    isem = (isem0, isem1, isem2)
    ssem = (ssem0, ssem1)

    def isrc(j):
        return dst_hbm.at[pl.ds((lo + j) * ECHUNK, ECHUNK)]

    pltpu.sync_copy(isrc(0), idx0)

    @pl.when(1 < n)
    def _():
        pltpu.async_copy(isrc(1), idx1, isem1)

    plsc.subcore_barrier()

    def body(j, _):
        for p2 in range(2):
            for p3 in range(3):
                @pl.when(jnp.logical_and(j % 2 == p2, j % 3 == p3))
                def _():
                    @pl.when(j >= 1)
                    def _():
                        pltpu.make_async_copy(
                            ones_v, acc.at[idx[(p3 + 2) % 3]],
                            ssem[1 - p2]).wait()

                    @pl.when(jnp.logical_and(j >= 1, j < n))
                    def _():
                        pltpu.make_async_copy(isrc(j), idx[p3],
                                              isem[p3]).wait()

                    pltpu.async_copy(ones_v, acc.at[idx[p3]], ssem[p2],
                                     add=True)

                    @pl.when(j + 2 < n)
                    def _():
                        pltpu.async_copy(isrc(j + 2), idx[(p3 + 2) % 3],
                                         isem[(p3 + 2) % 3])
        return _
    lax.fori_loop(0, n, body, None)

    for p2 in range(2):
        for p3 in range(3):
            @pl.when(jnp.logical_and((n - 1) % 2 == p2, (n - 1) % 3 == p3))
            def _():
                pltpu.make_async_copy(
                    ones_v, acc.at[idx[p3]], ssem[p2]).wait()

    plsc.subcore_barrier()
    _write_out(c, s, acc, out_hbm)


@functools.partial(
    pl.kernel,
    mesh=_MESH,
    out_type=jax.ShapeDtypeStruct((NUM_CORES, N_NODES, H), jnp.float32),
    scratch_types=[
        pltpu.VMEM((2, ECHUNK), jnp.int32),
        pltpu.VMEM((2, ECHUNK), jnp.int32),
        pltpu.VMEM((2, ECHUNK), jnp.int32),
        pltpu.VMEM((ECHUNK, H), jnp.float32),
        pltpu.VMEM((ECHUNK, H), jnp.float32),
        pltpu.VMEM_SHARED((N_NODES, H), jnp.float32),
        pltpu.SemaphoreType.DMA,
        pltpu.SemaphoreType.DMA,
        pltpu.SemaphoreType.DMA,
        pltpu.SemaphoreType.DMA,
        pltpu.SemaphoreType.DMA,
        pltpu.SemaphoreType.DMA,
        pltpu.SemaphoreType.DMA,
    ],
)
def _sc_edge_sum(g_hbm, edge_hbm, zeros_hbm, out_hbm,
                 idx0, idx1, idx2, rows0, rows1, acc,
                 isem0, isem1, isem2, gsem0, gsem1, ssem0, ssem1):
    c = lax.axis_index("c")
    s = lax.axis_index("s")
    t = c * NUM_SUBCORES + s

    _zero_acc(s, zeros_hbm, acc)

    lo = (t * NUM_ECHUNKS) // NUM_TILES
    hi = ((t + 1) * NUM_ECHUNKS) // NUM_TILES
    n = hi - lo

    idx = (idx0, idx1, idx2)
    rows = (rows0, rows1)
    isem = (isem0, isem1, isem2)
    gsem = (gsem0, gsem1)
    ssem = (ssem0, ssem1)

    def idx_src(j):
        return edge_hbm.at[:, pl.ds((lo + j) * ECHUNK, ECHUNK)]

    pltpu.sync_copy(idx_src(0), idx0)
    pltpu.async_copy(g_hbm.at[idx0.at[0]], rows0, gsem0)

    @pl.when(1 < n)
    def _():
        pltpu.async_copy(idx_src(1), idx1, isem1)

    plsc.subcore_barrier()

    def body(j, _):
        for p2 in range(2):
            for p3 in range(3):
                @pl.when(jnp.logical_and(j % 2 == p2, j % 3 == p3))
                def _():
                    @pl.when(j >= 1)
                    def _():
                        pltpu.make_async_copy(
                            rows[1 - p2], acc.at[idx[(p3 + 2) % 3].at[1]],
                            ssem[1 - p2]).wait()

                    @pl.when(j + 1 < n)
                    def _():
                        pltpu.make_async_copy(
                            idx_src(j + 1), idx[(p3 + 1) % 3],
                            isem[(p3 + 1) % 3]).wait()
                        pltpu.async_copy(
                            g_hbm.at[idx[(p3 + 1) % 3].at[0]],
                            rows[1 - p2], gsem[1 - p2])

                    pltpu.make_async_copy(
                        g_hbm.at[idx[p3].at[0]], rows[p2], gsem[p2]).wait()
                    pltpu.async_copy(
                        rows[p2], acc.at[idx[p3].at[1]], ssem[p2], add=True)

                    @pl.when(j + 2 < n)
                    def _():
                        pltpu.async_copy(idx_src(j + 2), idx[(p3 + 2) % 3],
                                         isem[(p3 + 2) % 3])
        return _

    lax.fori_loop(0, n, body, None)

    for p2 in range(2):
        for p3 in range(3):
            @pl.when(jnp.logical_and((n - 1) % 2 == p2, (n - 1) % 3 == p3))
            def _():
                pltpu.make_async_copy(
                    rows[p2], acc.at[idx[p3].at[1]], ssem[p2]).wait()

    plsc.subcore_barrier()
    _write_out(c, s, acc, out_hbm)


_R = 1000


def _tc_dense_body(x_ref, w0_ref, mlpw_ref, mlpb_ref, pw1w_ref, pw1b_ref,
                   h0_ref, proj1_ref):
    x = x_ref[...]
    h0_ref[...] = jnp.dot(x, w0_ref[...], preferred_element_type=jnp.float32)
    r1 = jnp.maximum(
        jnp.dot(x, mlpw_ref[...], preferred_element_type=jnp.float32)
        + mlpb_ref[...], 0.0)
    t = jnp.dot(r1, pw1w_ref[...], preferred_element_type=jnp.float32) + pw1b_ref[...]
    proj1_ref[...] = jnp.where(t > 0, t, ALPHA * t)


def _tc_scale_body(degp_ref, h0_ref, g0_ref, dinv_ref):
    deg = degp_ref[0, :, 0] + degp_ref[1, :, 0] + 1.0
    dinv = lax.rsqrt(jnp.maximum(deg, 1.0))[:, None]
    dinv_ref[...] = dinv
    g0_ref[...] = h0_ref[...] * dinv


def _tc_mid_body(p_ref, g_ref, dinv_ref, b_ref, w_ref, gout_ref):
    dinv = dinv_ref[...]
    h = jnp.maximum((p_ref[0] + p_ref[1] + g_ref[...]) * dinv + b_ref[...], 0.0)
    gout_ref[...] = jnp.dot(h, w_ref[...], preferred_element_type=jnp.float32) * dinv


def _tc_final_body(p_ref, g_ref, dinv_ref, b_ref, pw2w_ref, pw2b_ref,
                   proj1_ref, out_ref):
    dinv = dinv_ref[...]
    rna2d = jnp.maximum((p_ref[0] + p_ref[1] + g_ref[...]) * dinv + b_ref[...], 0.0)
    t = jnp.dot(rna2d, pw2w_ref[...], preferred_element_type=jnp.float32) + pw2b_ref[...]
    proj2 = jnp.where(t > 0, t, ALPHA * t)
    out_ref[...] = FRAC * proj2 + (1.0 - FRAC) * proj1_ref[...]


def _row_spec(width):
    return pl.BlockSpec((_R, width), lambda i: (i, 0))


def _full_spec(shape):
    nd = len(shape)
    return pl.BlockSpec(shape, lambda i: (0,) * nd)


def _part_spec(width):
    return pl.BlockSpec((NUM_CORES, _R, width), lambda i: (0, i, 0))


_GRID = (N_NODES // _R,)


def _tc_dense(x, w0, mlpw, mlpb, pw1w, pw1b):
    return pl.pallas_call(
        _tc_dense_body,
        grid=_GRID,
        in_specs=[
            _row_spec(D_IN),
            _full_spec((D_IN, H)),
            _full_spec((D_IN, H)),
            _full_spec((1, H)),
            _full_spec((H, H)),
            _full_spec((1, H)),
        ],
        out_specs=[_row_spec(H), _row_spec(H)],
        out_shape=[
            jax.ShapeDtypeStruct((N_NODES, H), jnp.float32),
            jax.ShapeDtypeStruct((N_NODES, H), jnp.float32),
        ],
    )(x, w0, mlpw, mlpb, pw1w, pw1b)


def _tc_scale(degp, h0):
    return pl.pallas_call(
        _tc_scale_body,
        grid=_GRID,
        in_specs=[_part_spec(H), _row_spec(H)],
        out_specs=[_row_spec(H), _row_spec(1)],
        out_shape=[
            jax.ShapeDtypeStruct((N_NODES, H), jnp.float32),
            jax.ShapeDtypeStruct((N_NODES, 1), jnp.float32),
        ],
    )(degp, h0)


def _tc_mid(p, g, dinv, b, w):
    return pl.pallas_call(
        _tc_mid_body,
        grid=_GRID,
        in_specs=[
            _part_spec(H),
            _row_spec(H),
            _row_spec(1),
            _full_spec((1, H)),
            _full_spec((H, H)),
        ],
        out_specs=_row_spec(H),
        out_shape=jax.ShapeDtypeStruct((N_NODES, H), jnp.float32),
    )(p, g, dinv, b, w)


def _tc_final(p, g, dinv, b, pw2w, pw2b, proj1):
    return pl.pallas_call(
        _tc_final_body,
        grid=_GRID,
        in_specs=[
            _part_spec(H),
            _row_spec(H),
            _row_spec(1),
            _full_spec((1, H)),
            _full_spec((H, H)),
            _full_spec((1, H)),
            _row_spec(H),
        ],
        out_specs=_row_spec(H),
        out_shape=jax.ShapeDtypeStruct((N_NODES, H), jnp.float32),
    )(p, g, dinv, b, pw2w, pw2b, proj1)


def kernel(x, gcn_W0, gcn_b0, gcn_W1, gcn_b1, gcn_W2, gcn_b2,
           mlp_W, mlp_b, pw2_W, pw2_b, pw1_W, pw1_b, edge_index):
    edge = edge_index.astype(jnp.int32)
    dst = edge[1]

    zerosH = jnp.zeros((ROWS_PER_SUBCORE, H), jnp.float32)
    onesH = jnp.ones((ECHUNK, H), jnp.float32)

    degp = _sc_degree(dst, zerosH, onesH)
    h0, proj1 = _tc_dense(
        x, gcn_W0, mlp_W, mlp_b.reshape(1, H), pw1_W, pw1_b.reshape(1, H))
    g0, dinv = _tc_scale(degp, h0)

    p0 = _sc_edge_sum(g0, edge, zerosH)
    g1 = _tc_mid(p0, g0, dinv, gcn_b0.reshape(1, H), gcn_W1)
    p1 = _sc_edge_sum(g1, edge, zerosH)
    g2 = _tc_mid(p1, g1, dinv, gcn_b1.reshape(1, H), gcn_W2)
    p2 = _sc_edge_sum(g2, edge, zerosH)

    return _tc_final(p2, g2, dinv, gcn_b2.reshape(1, H), pw2_W,
                     pw2_b.reshape(1, H), proj1)
